# Initial kernel scaffold; baseline (speedup 1.0000x reference)
import jax
import jax.numpy as jnp
from jax.experimental import pallas as pl


def kernel(x_nchw, weight_oihw, bias):
    raise NotImplementedError("write your pallas kernel here")


# speedup vs baseline: 1.2036x; 1.2036x over previous
import functools

import jax
import jax.numpy as jnp
from jax import lax
from jax.experimental import pallas as pl
from jax.experimental.pallas import tpu as pltpu

_TAPS = {(0, -1): (0,), (0, 0): (1, 2), (1, 0): (0, 1), (1, 1): (2,)}
_SHIFTS = ((-1, 0), (0, 1))


def _upconv_kernel(x_ref, w_ref, b_ref, o_ref, *, width, tile_h):
    x = x_ref[0]
    cin, L = x.shape
    cout = w_ref.shape[1]

    lane = lax.broadcasted_iota(jnp.int32, (cin, L), 1)
    col = lane % width
    zero = jnp.zeros_like(x)
    row_first = lane < width
    row_last = lane >= L - width
    col_first = col == 0
    col_last = col == width - 1

    r_up = jnp.where(row_first, zero, pltpu.roll(x, width, 1))
    r_dn = jnp.where(row_last, zero, pltpu.roll(x, L - width, 1))
    bases = {-1: r_up, 0: x, 1: r_dn}

    slabs = {}
    for a, base in bases.items():
        slabs[(a, -1)] = jnp.where(col_first, zero, pltpu.roll(base, 1, 1))
        slabs[(a, 0)] = base
        slabs[(a, 1)] = jnp.where(col_last, zero, pltpu.roll(base, L - 1, 1))

    bias = b_ref[...]
    idx = 0
    for py in (0, 1):
        for px in (0, 1):
            acc = None
            for a in _SHIFTS[py]:
                for b in _SHIFTS[px]:
                    contrib = jnp.dot(w_ref[idx], slabs[(a, b)],
                                      preferred_element_type=jnp.float32)
                    acc = contrib if acc is None else acc + contrib
                    idx += 1
            p = 2 * py + px
            acc = jnp.maximum(acc + bias, 0.0)
            o_ref[0, p * cout:(p + 1) * cout, :] = acc.astype(o_ref.dtype)


def _fold_weights(weight_oihw):
    blocks = []
    for py in (0, 1):
        for px in (0, 1):
            for a in _SHIFTS[py]:
                for b in _SHIFTS[px]:
                    w_sum = None
                    for kh in _TAPS[(py, a)]:
                        for kw in _TAPS[(px, b)]:
                            t = weight_oihw[:, :, kh, kw]
                            w_sum = t if w_sum is None else w_sum + t
                    blocks.append(w_sum)
    return jnp.stack(blocks, axis=0)


def kernel(x_nchw, weight_oihw, bias):
    N, Cin, H, W = x_nchw.shape
    Cout = weight_oihw.shape[0]
    out_dtype = x_nchw.dtype
    L = H * W

    w16 = _fold_weights(weight_oihw).astype(jnp.bfloat16)
    b2d = bias.reshape(Cout, 1).astype(jnp.float32)
    x_flat = x_nchw.reshape(N, Cin, L).astype(jnp.bfloat16)

    _kfn = functools.partial(_upconv_kernel, width=W, tile_h=H)

    out_k = pl.pallas_call(
        _kfn,
        out_shape=jax.ShapeDtypeStruct((N, 4 * Cout, L), out_dtype),
        grid=(N,),
        in_specs=[
            pl.BlockSpec((1, Cin, L), lambda n: (n, 0, 0)),
            pl.BlockSpec((16, Cout, Cin), lambda n: (0, 0, 0)),
            pl.BlockSpec((Cout, 1), lambda n: (0, 0)),
        ],
        out_specs=pl.BlockSpec((1, 4 * Cout, L), lambda n: (n, 0, 0)),
        compiler_params=pltpu.CompilerParams(
            dimension_semantics=("parallel",),
            vmem_limit_bytes=64 * 1024 * 1024),
    )(x_flat, w16, b2d)

    out = out_k.reshape(N, 2, 2, Cout, H, W)
    out = jnp.transpose(out, (0, 3, 4, 1, 5, 2))
    return out.reshape(N, Cout, 2 * H, 2 * W)


# speedup vs baseline: 2.3537x; 1.9555x over previous
import functools

import jax
import jax.numpy as jnp
from jax import lax
from jax.experimental import pallas as pl
from jax.experimental.pallas import tpu as pltpu


def _upconv_kernel(x_ref, p_ref, w_ref, b_ref, o_ref, *, width, height):
    xb = x_ref[0].astype(jnp.bfloat16)
    cin, L = xb.shape
    group = p_ref.shape[0] // width
    gl = group * width
    w2 = 2 * width
    L4 = 4 * L

    parts = [
        jnp.dot(xb[:, g * gl:(g + 1) * gl], p_ref[...],
                preferred_element_type=jnp.float32).astype(jnp.bfloat16)
        for g in range(L // gl)
    ]
    up = jnp.concatenate(parts, axis=1)

    lane = lax.broadcasted_iota(jnp.int32, (cin, L4), 1)
    colv = lane % w2
    zero = jnp.zeros_like(up)
    row_first = lane < w2
    row_last = lane >= L4 - w2
    col_first = colv == 0
    col_last = colv == w2 - 1

    acc = None
    for kh in (0, 1, 2):
        if kh == 0:
            base = jnp.where(row_first, zero, pltpu.roll(up, w2, 1))
        elif kh == 1:
            base = up
        else:
            base = jnp.where(row_last, zero, pltpu.roll(up, L4 - w2, 1))
        for kw in (0, 1, 2):
            if kw == 0:
                slab = jnp.where(col_first, zero, pltpu.roll(base, 1, 1))
            elif kw == 1:
                slab = base
            else:
                slab = jnp.where(col_last, zero, pltpu.roll(base, L4 - 1, 1))
            contrib = jnp.dot(w_ref[3 * kh + kw], slab,
                              preferred_element_type=jnp.float32)
            acc = contrib if acc is None else acc + contrib

    acc = jnp.maximum(acc + b_ref[...], 0.0)
    o_ref[0] = acc.astype(o_ref.dtype)


def _dilation_matrix(width, group):
    m = jnp.arange(4 * group * width, dtype=jnp.int32)
    src = (m // (4 * width)) * width + (m % (2 * width)) // 2
    k = jnp.arange(group * width, dtype=jnp.int32)
    return (k[:, None] == src[None, :]).astype(jnp.bfloat16)


def kernel(x_nchw, weight_oihw, bias):
    N, Cin, H, W = x_nchw.shape
    Cout = weight_oihw.shape[0]
    out_dtype = x_nchw.dtype
    L = H * W
    group = min(8, H)

    w9 = jnp.transpose(weight_oihw, (2, 3, 0, 1)).reshape(9, Cout, Cin)
    w9 = w9.astype(jnp.bfloat16)
    b2d = bias.reshape(Cout, 1).astype(jnp.float32)
    x_flat = x_nchw.reshape(N, Cin, L)
    pmat = _dilation_matrix(W, group)

    _kfn = functools.partial(_upconv_kernel, width=W, height=H)

    out_k = pl.pallas_call(
        _kfn,
        out_shape=jax.ShapeDtypeStruct((N, Cout, 4 * L), out_dtype),
        grid=(N,),
        in_specs=[
            pl.BlockSpec((1, Cin, L), lambda n: (n, 0, 0)),
            pl.BlockSpec(pmat.shape, lambda n: (0, 0)),
            pl.BlockSpec((9, Cout, Cin), lambda n: (0, 0, 0)),
            pl.BlockSpec((Cout, 1), lambda n: (0, 0)),
        ],
        out_specs=pl.BlockSpec((1, Cout, 4 * L), lambda n: (n, 0, 0)),
        compiler_params=pltpu.CompilerParams(
            dimension_semantics=("parallel",),
            vmem_limit_bytes=60 * 1024 * 1024),
    )(x_flat, pmat, w9, b2d)

    return out_k.reshape(N, Cout, 2 * H, 2 * W)


# speedup vs baseline: 4.3154x; 1.8335x over previous
import functools

import jax
import jax.numpy as jnp
from jax import lax
from jax.experimental import pallas as pl
from jax.experimental.pallas import tpu as pltpu


def _upconv_kernel(x_ref, p_ref, w_ref, b_ref, o_ref, *, width, height,
                   chunk):
    xb = x_ref[0].astype(jnp.bfloat16)
    L, cin = xb.shape
    gl = p_ref.shape[1]
    w2 = 2 * width
    L4 = 4 * L

    parts = [
        jnp.dot(p_ref[...], xb[g * gl:(g + 1) * gl, :],
                preferred_element_type=jnp.float32).astype(jnp.bfloat16)
        for g in range(L // gl)
    ]
    up = jnp.concatenate(parts, axis=0)

    row = lax.broadcasted_iota(jnp.int32, (L4, cin), 0)
    colv = row % w2
    zero = jnp.zeros_like(up)
    row_first = row < w2
    row_last = row >= L4 - w2
    col_first = colv == 0
    col_last = colv == w2 - 1

    slabs = []
    for kh in (0, 1, 2):
        if kh == 0:
            base = jnp.where(row_first, zero, pltpu.roll(up, w2, 0))
        elif kh == 1:
            base = up
        else:
            base = jnp.where(row_last, zero, pltpu.roll(up, L4 - w2, 0))
        for kw in (0, 1, 2):
            if kw == 0:
                slab = jnp.where(col_first, zero, pltpu.roll(base, 1, 0))
            elif kw == 1:
                slab = base
            else:
                slab = jnp.where(col_last, zero, pltpu.roll(base, L4 - 1, 0))
            slabs.append(slab)

    bias = b_ref[...]
    for c in range(0, L4, chunk):
        acc = None
        for s in range(9):
            contrib = jnp.dot(slabs[s][c:c + chunk, :], w_ref[s],
                              preferred_element_type=jnp.float32)
            acc = contrib if acc is None else acc + contrib
        acc = jnp.maximum(acc + bias, 0.0)
        o_ref[0, c:c + chunk, :] = acc.astype(o_ref.dtype)


def _dilation_matrix(width, group):
    m = jnp.arange(4 * group * width, dtype=jnp.int32)
    src = (m // (4 * width)) * width + (m % (2 * width)) // 2
    k = jnp.arange(group * width, dtype=jnp.int32)
    return (src[:, None] == k[None, :]).astype(jnp.bfloat16)


def kernel(x_nchw, weight_oihw, bias):
    N, Cin, H, W = x_nchw.shape
    Cout = weight_oihw.shape[0]
    out_dtype = x_nchw.dtype
    L = H * W
    group = min(8, H)

    w9 = jnp.transpose(weight_oihw, (2, 3, 1, 0)).reshape(9, Cin, Cout)
    w9 = w9.astype(jnp.bfloat16)
    b2d = bias.reshape(1, Cout).astype(jnp.float32)
    x_rows = jnp.transpose(x_nchw, (0, 2, 3, 1)).reshape(N, L, Cin)
    pmat = _dilation_matrix(W, group)

    _kfn = functools.partial(_upconv_kernel, width=W, height=H,
                             chunk=min(1024, 4 * L))

    out_k = pl.pallas_call(
        _kfn,
        out_shape=jax.ShapeDtypeStruct((N, 4 * L, Cout), out_dtype),
        grid=(N,),
        in_specs=[
            pl.BlockSpec((1, L, Cin), lambda n: (n, 0, 0)),
            pl.BlockSpec(pmat.shape, lambda n: (0, 0)),
            pl.BlockSpec((9, Cin, Cout), lambda n: (0, 0, 0)),
            pl.BlockSpec((1, Cout), lambda n: (0, 0)),
        ],
        out_specs=pl.BlockSpec((1, 4 * L, Cout), lambda n: (n, 0, 0)),
        compiler_params=pltpu.CompilerParams(
            dimension_semantics=("parallel",),
            vmem_limit_bytes=60 * 1024 * 1024),
    )(x_rows, pmat, w9, b2d)

    out = out_k.reshape(N, 2 * H, 2 * W, Cout)
    return jnp.transpose(out, (0, 3, 1, 2))
